# Initial kernel scaffold; baseline (speedup 1.0000x reference)
import jax
import jax.numpy as jnp
from jax.experimental import pallas as pl


def kernel(x, w1, grouped_gemm_batch_sizes):
    raise NotImplementedError("write your pallas kernel here")


# speedup vs baseline: 2.2606x; 2.2606x over previous
import functools

import jax
import jax.numpy as jnp
from jax.experimental import pallas as pl
from jax.experimental.pallas import tpu as pltpu


_BN = 256


def _router_body(expert_map_ref, x_ref, w_ref, wout_ref, iout_ref):
    s = jnp.dot(x_ref[...], w_ref[0], preferred_element_type=jnp.float32)
    bn, l = s.shape
    iota = jax.lax.broadcasted_iota(jnp.int32, (bn, l), 1)

    m1 = jnp.max(s, axis=1, keepdims=True)
    i1 = jnp.min(jnp.where(s == m1, iota, l), axis=1)
    z = jnp.sum(jnp.exp(s - m1), axis=1)
    masked = jnp.where(iota == i1[:, None], -jnp.inf, s)
    m2 = jnp.max(masked, axis=1, keepdims=True)
    i2 = jnp.min(jnp.where(masked == m2, iota, l), axis=1)

    w1v = 1.0 / z
    w2v = jnp.exp(m2[:, 0] - m1[:, 0]) / z
    wout_ref[...] = jnp.stack([w1v, w2v], axis=1)
    iout_ref[...] = jnp.stack([i1, i2], axis=1).astype(jnp.int32)


@jax.jit
def kernel(x, w1, grouped_gemm_batch_sizes):
    n, h = x.shape
    e = grouped_gemm_batch_sizes.shape[0]
    l = w1.shape[0] // e
    w = w1.reshape(e, h, l)
    num_blocks = n // _BN
    cum = jnp.cumsum(grouped_gemm_batch_sizes)
    starts = jnp.arange(num_blocks, dtype=jnp.int32) * _BN
    expert_map = jnp.searchsorted(cum, starts, side="right").astype(jnp.int32)

    grid_spec = pltpu.PrefetchScalarGridSpec(
        num_scalar_prefetch=1,
        grid=(num_blocks,),
        in_specs=[
            pl.BlockSpec((_BN, h), lambda i, m: (i, 0)),
            pl.BlockSpec((1, h, l), lambda i, m: (m[i], 0, 0)),
        ],
        out_specs=[
            pl.BlockSpec((_BN, 2), lambda i, m: (i, 0)),
            pl.BlockSpec((_BN, 2), lambda i, m: (i, 0)),
        ],
    )
    weights, indices = pl.pallas_call(
        _router_body,
        grid_spec=grid_spec,
        out_shape=[
            jax.ShapeDtypeStruct((n, 2), jnp.float32),
            jax.ShapeDtypeStruct((n, 2), jnp.int32),
        ],
    )(expert_map, x, w)
    return weights, indices
